# baseline (speedup 1.0000x reference)
import functools

import jax
import jax.numpy as jnp
from jax import lax
from jax.experimental import pallas as pl
from jax.experimental.pallas import tpu as pltpu
from jax.experimental.pallas import tpu_sc as plsc

B = 16384
D = 64
_INFO = plsc.get_sparse_core_info()
NC, NS, L = _INFO.num_cores, _INFO.num_subcores, _INFO.num_lanes
NW = NC * NS
CHUNK = B // NW
IDXW = 128
NSUB = CHUNK // IDXW
VPR = D // L

_sc_mesh = plsc.VectorSubcoreMesh(core_axis_name="c", subcore_axis_name="s")


@functools.partial(
    pl.kernel,
    mesh=_sc_mesh,
    out_type=jax.ShapeDtypeStruct((B, D), jnp.float32),
    scratch_types=[
        pltpu.VMEM((NSUB, IDXW), jnp.int32),
        pltpu.VMEM((CHUNK, D), jnp.float32),
        pltpu.VMEM((CHUNK, D), jnp.float32),
        pltpu.SemaphoreType.DMA,
    ],
    compiler_params=pltpu.CompilerParams(use_tc_tiling_on_sc=False),
)
def _sc_gather_sum(s_idx, i_idx, a_idx, m_idx,
                   s_tab, i_tab, a_tab, m_tab,
                   out, idx_v, acc_v, rows_v, sem):
    wid = lax.axis_index("s") * NC + lax.axis_index("c")
    base = wid * NSUB

    def gather_field(idx_hbm, tab_hbm, dst):
        pltpu.sync_copy(idx_hbm.at[pl.ds(base, NSUB)], idx_v)
        for j in range(NSUB):
            pltpu.async_copy(tab_hbm.at[idx_v.at[j]],
                             dst.at[pl.ds(j * IDXW, IDXW)], sem).wait()

    gather_field(s_idx, s_tab, acc_v)
    for idx_hbm, tab_hbm in ((i_idx, i_tab), (a_idx, a_tab), (m_idx, m_tab)):
        gather_field(idx_hbm, tab_hbm, rows_v)

        def body(r, carry):
            for c in range(VPR):
                sl = pl.ds(c * L, L)
                acc_v[r, sl] = acc_v[r, sl] + rows_v[r, sl]
            return carry
        lax.fori_loop(0, CHUNK, body, 0)

    pltpu.sync_copy(acc_v, out.at[pl.ds(wid * CHUNK, CHUNK)])


def _tc_project(idx_ref, comb_ref, w_ref, t0_ref, out_ref):
    mf = (idx_ref[...] == 0).astype(jnp.float32)
    t0w = jnp.dot(t0_ref[...], w_ref[...],
                  preferred_element_type=jnp.float32)
    main = jnp.dot(comb_ref[...], w_ref[...],
                   preferred_element_type=jnp.float32)
    corr = lax.dot_general(mf, t0w, (((0,), (0,)), ((), ())),
                           preferred_element_type=jnp.float32)
    out_ref[...] = main - corr


def kernel(species_idx, item_idx, ability_idx, move_idx,
           species_table, items_table, abilities_table, moves_table, W):
    idx = [a.astype(jnp.int32)
           for a in (species_idx, item_idx, ability_idx, move_idx)]
    tabs = (species_table, items_table, abilities_table, moves_table)

    idx2d = [a.reshape(B // IDXW, IDXW) for a in idx]
    combined = _sc_gather_sum(*idx2d, *tabs)

    idx8 = jnp.concatenate(
        [jnp.stack(idx), jnp.ones((4, B), jnp.int32)], axis=0)
    t08 = jnp.concatenate(
        [jnp.stack([t[0] for t in tabs]),
         jnp.zeros((4, D), jnp.float32)], axis=0)

    TB = 2048
    out = pl.pallas_call(
        _tc_project,
        grid=(B // TB,),
        in_specs=[
            pl.BlockSpec((8, TB), lambda i: (0, i)),
            pl.BlockSpec((TB, D), lambda i: (i, 0)),
            pl.BlockSpec((D, D), lambda i: (0, 0)),
            pl.BlockSpec((8, D), lambda i: (0, 0)),
        ],
        out_specs=pl.BlockSpec((TB, D), lambda i: (i, 0)),
        out_shape=jax.ShapeDtypeStruct((B, D), jnp.float32),
    )(idx8, combined, W, t08)
    return out


# speedup vs baseline: 1.0410x; 1.0410x over previous
import functools

import jax
import jax.numpy as jnp
from jax import lax
from jax.experimental import pallas as pl
from jax.experimental.pallas import tpu as pltpu
from jax.experimental.pallas import tpu_sc as plsc

B = 16384
D = 64
_INFO = plsc.get_sparse_core_info()
NC, NS, L = _INFO.num_cores, _INFO.num_subcores, _INFO.num_lanes
NW = NC * NS
CHUNK = B // NW
IDXW = 128
NSUB = CHUNK // IDXW
VPR = D // L

_sc_mesh = plsc.VectorSubcoreMesh(core_axis_name="c", subcore_axis_name="s")


@functools.partial(
    pl.kernel,
    mesh=_sc_mesh,
    out_type=jax.ShapeDtypeStruct((B, D), jnp.float32),
    scratch_types=[
        pltpu.VMEM((NSUB, IDXW), jnp.int32),
        pltpu.VMEM((CHUNK,), jnp.int32),
        pltpu.VMEM((CHUNK, D), jnp.float32),
        pltpu.VMEM_SHARED((NS * CHUNK, D), jnp.float32),
        pltpu.SemaphoreType.DMA,
    ],
    compiler_params=pltpu.CompilerParams(use_tc_tiling_on_sc=False),
)
def _sc_gather_sum(s_idx, i_idx, a_idx, m_idx,
                   s_tab, i_tab, a_tab, m_tab,
                   out, idx_v, ids_v, rows_v, acc_sh, sem):
    sid = lax.axis_index("s")
    wid = sid * NC + lax.axis_index("c")
    base = wid * NSUB
    slab = sid * CHUNK

    for g in range(CHUNK // L):
        ids_v[pl.ds(g * L, L)] = lax.iota(jnp.int32, L) + (slab + g * L)

    def gather_field(idx_hbm, tab_hbm):
        pltpu.sync_copy(idx_hbm.at[pl.ds(base, NSUB)], idx_v)
        cps = [pltpu.async_copy(tab_hbm.at[idx_v.at[j]],
                                rows_v.at[pl.ds(j * IDXW, IDXW)], sem)
               for j in range(NSUB)]
        for cp in cps:
            cp.wait()

    gather_field(s_idx, s_tab)
    pltpu.sync_copy(rows_v, acc_sh.at[pl.ds(slab, CHUNK)])
    for idx_hbm, tab_hbm in ((i_idx, i_tab), (a_idx, a_tab), (m_idx, m_tab)):
        gather_field(idx_hbm, tab_hbm)
        pltpu.sync_copy(rows_v, acc_sh.at[ids_v], add=True)

    pltpu.sync_copy(acc_sh.at[pl.ds(slab, CHUNK)],
                    out.at[pl.ds(wid * CHUNK, CHUNK)])


def _tc_project(idx_ref, comb_ref, w_ref, t0_ref, out_ref):
    mf = (idx_ref[...] == 0).astype(jnp.float32)
    t0w = jnp.dot(t0_ref[...], w_ref[...],
                  preferred_element_type=jnp.float32)
    main = jnp.dot(comb_ref[...], w_ref[...],
                   preferred_element_type=jnp.float32)
    corr = lax.dot_general(mf, t0w, (((0,), (0,)), ((), ())),
                           preferred_element_type=jnp.float32)
    out_ref[...] = main - corr


def kernel(species_idx, item_idx, ability_idx, move_idx,
           species_table, items_table, abilities_table, moves_table, W):
    idx = [a.astype(jnp.int32)
           for a in (species_idx, item_idx, ability_idx, move_idx)]
    tabs = (species_table, items_table, abilities_table, moves_table)

    idx2d = [a.reshape(B // IDXW, IDXW) for a in idx]
    combined = _sc_gather_sum(*idx2d, *tabs)

    idx8 = jnp.concatenate(
        [jnp.stack(idx), jnp.ones((4, B), jnp.int32)], axis=0)
    t08 = jnp.concatenate(
        [jnp.stack([t[0] for t in tabs]),
         jnp.zeros((4, D), jnp.float32)], axis=0)

    TB = 4096
    out = pl.pallas_call(
        _tc_project,
        grid=(B // TB,),
        in_specs=[
            pl.BlockSpec((8, TB), lambda i: (0, i)),
            pl.BlockSpec((TB, D), lambda i: (i, 0)),
            pl.BlockSpec((D, D), lambda i: (0, 0)),
            pl.BlockSpec((8, D), lambda i: (0, 0)),
        ],
        out_specs=pl.BlockSpec((TB, D), lambda i: (i, 0)),
        out_shape=jax.ShapeDtypeStruct((B, D), jnp.float32),
    )(idx8, combined, W, t08)
    return out


# speedup vs baseline: 1.0566x; 1.0150x over previous
import functools

import jax
import jax.numpy as jnp
from jax import lax
from jax.experimental import pallas as pl
from jax.experimental.pallas import tpu as pltpu
from jax.experimental.pallas import tpu_sc as plsc

B = 16384
D = 64
_INFO = plsc.get_sparse_core_info()
NC, NS, L = _INFO.num_cores, _INFO.num_subcores, _INFO.num_lanes
NW = NC * NS
CHUNK = B // NW
IDXW = 128
NSUB = CHUNK // IDXW

_sc_mesh = plsc.VectorSubcoreMesh(core_axis_name="c", subcore_axis_name="s")


@functools.partial(
    pl.kernel,
    mesh=_sc_mesh,
    out_type=jax.ShapeDtypeStruct((B, D), jnp.float32),
    scratch_types=[
        pltpu.VMEM((4 * NSUB, IDXW), jnp.int32),
        pltpu.VMEM((CHUNK,), jnp.int32),
        pltpu.VMEM((CHUNK, D), jnp.float32),
        pltpu.VMEM((CHUNK, D), jnp.float32),
        pltpu.VMEM_SHARED((NS * CHUNK, D), jnp.float32),
        pltpu.SemaphoreType.DMA,
        pltpu.SemaphoreType.DMA,
        pltpu.SemaphoreType.DMA,
    ],
    compiler_params=pltpu.CompilerParams(use_tc_tiling_on_sc=False),
)
def _sc_gather_sum(s_idx, i_idx, a_idx, m_idx,
                   s_tab, i_tab, a_tab, m_tab,
                   out, idx_v, ids_v, rows_a, rows_b, acc_sh,
                   sem_i, sem_a, sem_b):
    sid = lax.axis_index("s")
    wid = sid * NC + lax.axis_index("c")
    base = wid * NSUB
    slab = sid * CHUNK

    idx_in = (s_idx, i_idx, a_idx, m_idx)
    icps = [pltpu.async_copy(idx_in[f].at[pl.ds(base, NSUB)],
                             idx_v.at[pl.ds(f * NSUB, NSUB)], sem_i)
            for f in range(4)]

    for g in range(CHUNK // L):
        ids_v[pl.ds(g * L, L)] = lax.iota(jnp.int32, L) + (slab + g * L)

    for cp in icps:
        cp.wait()

    tabs = (s_tab, i_tab, a_tab, m_tab)
    bufs = (rows_a, rows_b)
    sems = (sem_a, sem_b)

    def fire(f):
        tab, buf, sem = tabs[f], bufs[f % 2], sems[f % 2]
        return [pltpu.async_copy(tab.at[idx_v.at[f * NSUB + j]],
                                 buf.at[pl.ds(j * IDXW, IDXW)], sem)
                for j in range(NSUB)]

    pend = fire(0)
    for f in range(1, 4):
        nxt = fire(f)
        for cp in pend:
            cp.wait()
        buf = bufs[(f - 1) % 2]
        if f == 1:
            pltpu.sync_copy(buf, acc_sh.at[pl.ds(slab, CHUNK)])
        else:
            pltpu.sync_copy(buf, acc_sh.at[ids_v], add=True)
        pend = nxt
    for cp in pend:
        cp.wait()
    pltpu.sync_copy(rows_b, acc_sh.at[ids_v], add=True)

    pltpu.sync_copy(acc_sh.at[pl.ds(slab, CHUNK)],
                    out.at[pl.ds(wid * CHUNK, CHUNK)])


def _tc_project(idx_ref, comb_ref, w_ref, t0_ref, out_ref):
    mf = (idx_ref[...] == 0).astype(jnp.float32)
    t0w = jnp.dot(t0_ref[...], w_ref[...],
                  preferred_element_type=jnp.float32)
    main = jnp.dot(comb_ref[...], w_ref[...],
                   preferred_element_type=jnp.float32)
    corr = lax.dot_general(mf, t0w, (((0,), (0,)), ((), ())),
                           preferred_element_type=jnp.float32)
    out_ref[...] = main - corr


def kernel(species_idx, item_idx, ability_idx, move_idx,
           species_table, items_table, abilities_table, moves_table, W):
    idx = [a.astype(jnp.int32)
           for a in (species_idx, item_idx, ability_idx, move_idx)]
    tabs = (species_table, items_table, abilities_table, moves_table)

    idx2d = [a.reshape(B // IDXW, IDXW) for a in idx]
    combined = _sc_gather_sum(*idx2d, *tabs)

    idx8 = jnp.concatenate(
        [jnp.stack(idx), jnp.ones((4, B), jnp.int32)], axis=0)
    t08 = jnp.concatenate(
        [jnp.stack([t[0] for t in tabs]),
         jnp.zeros((4, D), jnp.float32)], axis=0)

    TB = 4096
    out = pl.pallas_call(
        _tc_project,
        grid=(B // TB,),
        in_specs=[
            pl.BlockSpec((8, TB), lambda i: (0, i)),
            pl.BlockSpec((TB, D), lambda i: (i, 0)),
            pl.BlockSpec((D, D), lambda i: (0, 0)),
            pl.BlockSpec((8, D), lambda i: (0, 0)),
        ],
        out_specs=pl.BlockSpec((TB, D), lambda i: (i, 0)),
        out_shape=jax.ShapeDtypeStruct((B, D), jnp.float32),
    )(idx8, combined, W, t08)
    return out
